# baseline (speedup 1.0000x reference)
import functools

import jax
import jax.numpy as jnp
from jax import lax
from jax.experimental import pallas as pl
from jax.experimental.pallas import tpu as pltpu
import jax.experimental.pallas.tpu_sc as plsc

NB = 10000
NF = 40000
NTOT = NB + NF
C = 256
ND = 64
NM = 8192
E = 300000

NC = 2
NS = 16
NW = NC * NS

ROWS_W = 1280
NFPAD = ROWS_W * NW
GB = 128
NGB = ROWS_W // GB
TROWS = NB + NFPAD + 16
ZROW = NB + NFPAD
XCHUNK = NB // NW

STW = 256
NBLK2 = 148
EPAD = NBLK2 * 8 * STW
RACC = 392
NPASS = 4
YROWS = NPASS * NW * RACC
DB = 80
CAP = 81

_mesh = plsc.VectorSubcoreMesh(core_axis_name="c", subcore_axis_name="s")


@functools.partial(
    pl.kernel,
    out_type=(
        jax.ShapeDtypeStruct((TROWS, C), jnp.float32),
        jax.ShapeDtypeStruct((NFPAD, C), jnp.float32),
    ),
    mesh=_mesh,
    scratch_types=[
        pltpu.VMEM((ROWS_W,), jnp.int32),
        pltpu.VMEM((GB,), jnp.int32),
        pltpu.VMEM((GB, 2 * ND), jnp.float32),
        pltpu.VMEM((GB, C), jnp.float32),
        pltpu.VMEM((GB, C), jnp.float32),
        pltpu.SemaphoreType.DMA,
    ],
)
def _gather_k(x_hbm, fidx_hbm, c0, c1, c2, c3, cb0, cb1, cb2, cb3,
              t_hbm, g_hbm, fidx_v, foc_v, ra, xc, gc, sem):
    wid = lax.axis_index("c") * NS + lax.axis_index("s")
    base = pl.multiple_of(wid * ROWS_W, 128)
    pltpu.sync_copy(fidx_hbm.at[pl.ds(base, ROWS_W)], fidx_v)

    def body(b, _):
        for i, (ci, cbi) in enumerate(
                ((c0, cb0), (c1, cb1), (c2, cb2), (c3, cb3))):
            pltpu.async_copy(
                ci.at[fidx_v.at[pl.ds(pl.multiple_of(b * GB, 128), GB)]],
                foc_v, sem).wait()
            pltpu.async_copy(cbi.at[foc_v], ra, sem).wait()

            def asm(r, _, i=i):
                for k in range(ND // 16):
                    xc[r, pl.ds(i * ND + k * 16, 16)] = ra[r, pl.ds(k * 16, 16)]
                    gc[r, pl.ds(i * ND + k * 16, 16)] = ra[r, pl.ds(ND + k * 16, 16)]
                return ()
            lax.fori_loop(0, GB, asm, ())

        r0 = pl.multiple_of(base + b * GB, 128)
        pltpu.sync_copy(xc, t_hbm.at[pl.ds(NB + r0, GB)])
        pltpu.sync_copy(gc, g_hbm.at[pl.ds(r0, GB)])
        return ()
    lax.fori_loop(0, NGB, body, ())

    r0 = pl.multiple_of(wid * XCHUNK, 8)
    pltpu.sync_copy(x_hbm.at[pl.ds(r0, XCHUNK)], t_hbm.at[pl.ds(r0, XCHUNK)])

    @pl.when(wid == 0)
    def _():
        zv = jnp.zeros((16,), jnp.float32)

        def zr(r, _):
            for k in range(C // 16):
                xc[r, pl.ds(k * 16, 16)] = zv
            return ()
        lax.fori_loop(0, 16, zr, ())
        rt = NW * XCHUNK
        pltpu.sync_copy(x_hbm.at[pl.ds(rt, NB - rt)], t_hbm.at[pl.ds(rt, NB - rt)])
        pltpu.sync_copy(xc.at[pl.ds(0, 16)], t_hbm.at[pl.ds(ZROW, 16)])


@functools.partial(
    pl.kernel,
    out_type=jax.ShapeDtypeStruct((YROWS, C), jnp.float32),
    mesh=_mesh,
    scratch_types=[
        pltpu.VMEM((8, STW), jnp.int32),
        pltpu.VMEM((8, STW), jnp.int32),
        pltpu.VMEM((CAP * 16,), jnp.int32),
        pltpu.VMEM((DB + 16,), jnp.int32),
        pltpu.VMEM((DB, C), jnp.float32),
        pltpu.VMEM((RACC + 8, C), jnp.float32),
        pltpu.SMEM((8,), jnp.int32),
        pltpu.SemaphoreType.DMA,
    ],
)
def _edge_k(t_hbm, src_hbm, dst_hbm, y_hbm,
            sv, dv, elist, sidx, rows_v, acc, cnt_s, sem):
    cid = lax.axis_index("c")
    sid = lax.axis_index("s")
    wid = cid * NS + sid
    zv = jnp.zeros((16,), jnp.float32)
    pad_entry = ZROW * 512 + RACC

    def drain_gated(flag01):
        def drA(_b, _2):
            def comp(e2, _3):
                ev = elist[pl.ds(e2 * 16, 16)][0]
                sidx[pl.ds(e2, 16)] = jnp.full((16,), ev >> 9, jnp.int32)
                return ()
            lax.fori_loop(0, DB, comp, ())
            pltpu.async_copy(t_hbm.at[sidx.at[pl.ds(0, DB)]], rows_v, sem).wait()
            return ()
        lax.fori_loop(0, flag01, drA, ())

        def drB(e, _2):
            ev = elist[pl.ds(e * 16, 16)][0]
            d = ev & 511
            for c in range(C // 16):
                acc[d, pl.ds(c * 16, 16)] = (
                    acc[d, pl.ds(c * 16, 16)] + rows_v[e, pl.ds(c * 16, 16)])
            return ()
        lax.fori_loop(0, flag01 * DB, drB, ())

        pv = jnp.full((16,), pad_entry, jnp.int32)

        def drC(i, _2):
            elist[pl.ds(i * 16, 16)] = pv
            return ()
        lax.fori_loop(0, flag01 * CAP, drC, ())
        cnt_s[0] = cnt_s[0] * (1 - flag01)

    for p in range(NPASS):
        rng = p * NW + wid
        lo = rng * RACC
        hi = lo + RACC

        def zacc(r, _):
            for c in range(C // 16):
                acc[r, pl.ds(c * 16, 16)] = zv
            return ()
        lax.fori_loop(0, RACC + 8, zacc, ())
        pv0 = jnp.full((16,), pad_entry, jnp.int32)

        def pad0(i, _):
            elist[pl.ds(i * 16, 16)] = pv0
            return ()
        lax.fori_loop(0, CAP, pad0, ())
        cnt_s[0] = 0

        def step(k, _, lo=lo, hi=hi):
            ld01 = ((k & 127) == 0).astype(jnp.int32)

            def ldb(_b, _2):
                e0 = pl.multiple_of((k >> 7) * 8, 8)
                pltpu.sync_copy(src_hbm.at[pl.ds(e0, 8), pl.ds(0, STW)], sv)
                pltpu.sync_copy(dst_hbm.at[pl.ds(e0, 8), pl.ds(0, STW)], dv)
                return ()
            lax.fori_loop(0, ld01, ldb, ())

            r = (k & 127) >> 4
            co = (k & 15) * 16
            dvv = dv[r, pl.ds(co, 16)].reshape(16)
            svv = sv[r, pl.ds(co, 16)].reshape(16)
            m = (dvv >= lo) & (dvv < hi)
            pk2 = jnp.where(m, svv * 512 + (dvv - lo), pad_entry)
            for l in range(16):
                pk = pk2[l]
                ml = (pk != pad_entry).astype(jnp.int32)
                c2 = cnt_s[0]
                off = c2 * ml + (CAP - 1) * (1 - ml)
                elist[pl.ds(off * 16, 16)] = jnp.full((16,), pk, jnp.int32)
                cnt_s[0] = c2 + ml
            cnt_s[5] = (cnt_s[0] >= DB - 16).astype(jnp.int32)
            drain_gated(cnt_s[5])
            return ()
        lax.fori_loop(0, EPAD // 16, step, ())

        cnt_s[5] = (cnt_s[0] > 0).astype(jnp.int32)
        drain_gated(cnt_s[5])

        pltpu.sync_copy(acc.at[pl.ds(0, RACC)],
                        y_hbm.at[pl.ds(pl.multiple_of(lo, 8), RACC)])


def _out_body(y_ref, x_ref, wc_ref, wg_ref, ws_ref, bg_ref, bs_ref, o_ref):
    hp = jax.lax.Precision.HIGHEST
    t = jnp.dot(y_ref[...], wc_ref[...], precision=hp,
                preferred_element_type=jnp.float32)
    t = jnp.dot(t, wg_ref[...], precision=hp,
                preferred_element_type=jnp.float32)
    t = t + jnp.dot(x_ref[...], ws_ref[...], precision=hp,
                    preferred_element_type=jnp.float32)
    o_ref[...] = t + bg_ref[...] + bs_ref[...]


def _info_body(y_ref, g_ref, wc_ref, wu_ref, o_ref):
    i = pl.program_id(0)
    hp = jax.lax.Precision.HIGHEST
    t = jnp.dot(y_ref[...], wc_ref[...], precision=hp,
                preferred_element_type=jnp.float32)
    part = jnp.sum(t * g_ref[...]) * wu_ref[0, 0]

    @pl.when(i == 0)
    def _():
        o_ref[...] = jnp.zeros((1, 1), jnp.float32)
    o_ref[...] = o_ref[...] + part


_BLK = 1000


def _tc_out(y, x, W_conv, W_gnn, W_skip, b_gnn, b_skip):
    full = pl.BlockSpec((C, C), lambda i: (0, 0))
    bias = pl.BlockSpec((1, C), lambda i: (0, 0))
    return pl.pallas_call(
        _out_body,
        grid=(NB // _BLK,),
        in_specs=[
            pl.BlockSpec((_BLK, C), lambda i: (i, 0)),
            pl.BlockSpec((_BLK, C), lambda i: (i, 0)),
            full, full, full, bias, bias,
        ],
        out_specs=pl.BlockSpec((_BLK, C), lambda i: (i, 0)),
        out_shape=jax.ShapeDtypeStruct((NB, C), jnp.float32),
    )(y, x, W_conv, W_gnn, W_skip, b_gnn.reshape(1, C), b_skip.reshape(1, C))


def _tc_info(y, g, W_conv, wu):
    full = pl.BlockSpec((C, C), lambda i: (0, 0))
    return pl.pallas_call(
        _info_body,
        grid=(NF // _BLK,),
        in_specs=[
            pl.BlockSpec((_BLK, C), lambda i: (i + NB // _BLK, 0)),
            pl.BlockSpec((_BLK, C), lambda i: (i, 0)),
            full,
            pl.BlockSpec((1, 1), lambda i: (0, 0)),
        ],
        out_specs=pl.BlockSpec((1, 1), lambda i: (0, 0)),
        out_shape=jax.ShapeDtypeStruct((1, 1), jnp.float32),
    )(y, g, W_conv, wu.reshape(1, 1))


@jax.jit
def kernel(x, batch_idx, subset, edge_index, warm_up_rate,
           c_indices, codebooks, W_conv, W_gnn, b_gnn, W_skip, b_skip):
    del batch_idx
    fidx = subset[NB:]
    fidx_pad = jnp.concatenate(
        [fidx, jnp.zeros((NFPAD - NF,), fidx.dtype)])
    t, g = _gather_k(
        x, fidx_pad,
        c_indices[0], c_indices[1], c_indices[2], c_indices[3],
        codebooks[0], codebooks[1], codebooks[2], codebooks[3])
    src = jnp.concatenate(
        [edge_index[0], jnp.full((EPAD - E,), ZROW, jnp.int32)]).reshape(
            NBLK2 * 8, STW)
    dst = jnp.concatenate(
        [edge_index[1], jnp.zeros((EPAD - E,), jnp.int32)]).reshape(
            NBLK2 * 8, STW)
    y = _edge_k(t, src, dst)
    out = _tc_out(y, x, W_conv, W_gnn, W_skip, b_gnn, b_skip)
    info = _tc_info(y, g, W_conv, warm_up_rate)
    return out, info[0, 0]
